# Initial kernel scaffold; baseline (speedup 1.0000x reference)
import jax
import jax.numpy as jnp
from jax.experimental import pallas as pl


def kernel(Target, Source, s_num):
    raise NotImplementedError("write your pallas kernel here")


# speedup vs baseline: 17.5149x; 17.5149x over previous
import functools

import jax
import jax.numpy as jnp
from jax.experimental import pallas as pl
from jax.experimental.pallas import tpu as pltpu

BT = 256
K = 4


def _topk_body(q_ref, s_ref, idx_ref):
    q = q_ref[0]
    s = s_ref[0]
    qn = jnp.sum(q * q, axis=0)
    sn = jnp.sum(s * s, axis=0)
    qs = jax.lax.dot_general(q, s, (((0,), (0,)), ((), ())),
                             preferred_element_type=jnp.float32)
    dist = (qn[:, None] + sn[None, :]) - 2.0 * qs
    ps = dist.shape[1]
    lane = jax.lax.broadcasted_iota(jnp.int32, dist.shape, 1)
    cols = []
    d = dist
    for _ in range(K):
        m = jnp.min(d, axis=1, keepdims=True)
        cand = jnp.where(d == m, lane, jnp.int32(ps))
        amin = jnp.min(cand, axis=1, keepdims=True)
        cols.append(amin)
        d = jnp.where(lane == amin, jnp.float32(jnp.inf), d)
    idx_ref[0] = jnp.concatenate(cols, axis=1)


def _knn_topk(Tq, S):
    B, d, PT = Tq.shape
    PS = S.shape[2]
    grid = (B, PT // BT)
    return pl.pallas_call(
        _topk_body,
        grid=grid,
        in_specs=[
            pl.BlockSpec((1, d, BT), lambda b, p: (b, 0, p)),
            pl.BlockSpec((1, d, PS), lambda b, p: (b, 0, 0)),
        ],
        out_specs=pl.BlockSpec((1, BT, K), lambda b, p: (b, p, 0)),
        out_shape=jax.ShapeDtypeStruct((B, PT, K), jnp.int32),
    )(Tq, S)


def kernel(Target, Source, s_num):
    B, C, PT = Target.shape
    idx = _knn_topk(Target[:, 4:, :], Source[:, 4:, :])
    idx = idx + (jnp.asarray(s_num, dtype=idx.dtype) - K)
    idx_b = jnp.broadcast_to(idx.reshape(B, 1, PT * K), (B, C, PT * K))
    feat = jnp.take_along_axis(Source, idx_b, axis=2)
    return feat.reshape(B, C, PT, K)


# speedup vs baseline: 45.8545x; 2.6180x over previous
import functools

import jax
import jax.numpy as jnp
from jax import lax
from jax.experimental import pallas as pl
from jax.experimental.pallas import tpu as pltpu
from jax.experimental.pallas import tpu_sc as plsc

BT = 256
K = 4


def _topk_body(q_ref, s_ref, idx_ref):
    q = q_ref[0]
    s = s_ref[0]
    qn = jnp.sum(q * q, axis=0)
    sn = jnp.sum(s * s, axis=0)
    qs = jax.lax.dot_general(q, s, (((0,), (0,)), ((), ())),
                             preferred_element_type=jnp.float32)
    dist = (qn[:, None] + sn[None, :]) - 2.0 * qs
    ps = dist.shape[1]
    lane = jax.lax.broadcasted_iota(jnp.int32, dist.shape, 1)
    cols = []
    d = dist
    for _ in range(K):
        m = jnp.min(d, axis=1, keepdims=True)
        cand = jnp.where(d == m, lane, jnp.int32(ps))
        amin = jnp.min(cand, axis=1, keepdims=True)
        cols.append(amin)
        d = jnp.where(lane == amin, jnp.float32(jnp.inf), d)
    idx_ref[0] = jnp.concatenate(cols, axis=1)


def _knn_topk(Tq, S):
    B, d, PT = Tq.shape
    PS = S.shape[2]
    grid = (B, PT // BT)
    return pl.pallas_call(
        _topk_body,
        grid=grid,
        in_specs=[
            pl.BlockSpec((1, d, BT), lambda b, p: (b, 0, p)),
            pl.BlockSpec((1, d, PS), lambda b, p: (b, 0, 0)),
        ],
        out_specs=pl.BlockSpec((1, BT, K), lambda b, p: (b, p, 0)),
        out_shape=jax.ShapeDtypeStruct((B, PT, K), jnp.int32),
    )(Tq, S)


def _sc_gather(idxf, Source):
    B, C, PS = Source.shape
    PTK = idxf.shape[1]
    ncores, nsub = 2, 16
    nw = ncores * nsub
    gpb = nw // B
    nch = -(-C // gpb)
    mesh = plsc.VectorSubcoreMesh(core_axis_name="c", subcore_axis_name="s")

    @functools.partial(
        pl.kernel,
        out_type=jax.ShapeDtypeStruct((B, C, PTK), jnp.float32),
        mesh=mesh,
        compiler_params=pltpu.CompilerParams(needs_layout_passes=False),
        scratch_types=[
            pltpu.VMEM((PTK,), jnp.int32),
            pltpu.VMEM((PS,), jnp.float32),
            pltpu.VMEM((PTK,), jnp.float32),
        ],
    )
    def gather_kernel(idx_hbm, src_hbm, out_hbm, idx_v, row_v, out_v):
        wid = lax.axis_index("s") * ncores + lax.axis_index("c")
        b = wid // gpb
        g = wid % gpb
        pltpu.sync_copy(idx_hbm.at[b], idx_v)

        def chan_body(i, carry):
            ch = g + i * gpb

            @pl.when(ch < C)
            def _():
                pltpu.sync_copy(src_hbm.at[b, ch], row_v)

                def vec_body(j, carry2):
                    base = j * 64
                    for u in range(4):
                        ivec = idx_v[pl.ds(base + u * 16, 16)]
                        out_v[pl.ds(base + u * 16, 16)] = plsc.load_gather(
                            row_v, [ivec])
                    return carry2

                lax.fori_loop(0, PTK // 64, vec_body, 0)
                pltpu.sync_copy(out_v, out_hbm.at[b, ch])
            return carry

        lax.fori_loop(0, nch, chan_body, 0)

    return gather_kernel(idxf, Source)


def kernel(Target, Source, s_num):
    B, C, PT = Target.shape
    idx = _knn_topk(Target[:, 4:, :], Source[:, 4:, :])
    idx = idx + (jnp.asarray(s_num, dtype=idx.dtype) - K)
    feat = _sc_gather(idx.reshape(B, PT * K), Source)
    return feat.reshape(B, C, PT, K)


# speedup vs baseline: 52.1301x; 1.1369x over previous
import functools

import jax
import jax.numpy as jnp
from jax import lax
from jax.experimental import pallas as pl
from jax.experimental.pallas import tpu as pltpu
from jax.experimental.pallas import tpu_sc as plsc

BT = 256
K = 4


def _topk_body(q_ref, s_ref, idx_ref):
    q = q_ref[0]
    s = s_ref[0]
    qn = jnp.sum(q * q, axis=0)
    sn = jnp.sum(s * s, axis=0)
    qs = jax.lax.dot_general(q, s, (((0,), (0,)), ((), ())),
                             preferred_element_type=jnp.float32)
    dist = (qn[:, None] + sn[None, :]) - 2.0 * qs
    ps = dist.shape[1]
    lane = jax.lax.broadcasted_iota(jnp.int32, dist.shape, 1).astype(jnp.float32)
    cols = []
    d = dist
    for j in range(K):
        m = jnp.min(d, axis=1, keepdims=True)
        cand = jnp.where(d == m, lane, jnp.float32(ps))
        amin = jnp.min(cand, axis=1, keepdims=True)
        cols.append(amin)
        if j < K - 1:
            d = jnp.where(lane == amin, jnp.float32(jnp.inf), d)
    idxf = jnp.concatenate(cols, axis=1)
    idx_ref[0] = idxf.astype(jnp.int32)


def _knn_topk(Tq, S):
    B, d, PT = Tq.shape
    PS = S.shape[2]
    grid = (B, PT // BT)
    return pl.pallas_call(
        _topk_body,
        grid=grid,
        in_specs=[
            pl.BlockSpec((1, d, BT), lambda b, p: (b, 0, p)),
            pl.BlockSpec((1, d, PS), lambda b, p: (b, 0, 0)),
        ],
        out_specs=pl.BlockSpec((1, BT, K), lambda b, p: (b, p, 0)),
        out_shape=jax.ShapeDtypeStruct((B, PT, K), jnp.int32),
    )(Tq, S)


def _sc_gather(idxf, Source):
    B, C, PS = Source.shape
    PTK = idxf.shape[1]
    ncores, nsub = 2, 16
    nw = ncores * nsub
    gpb = nw // B
    nch = -(-C // gpb)
    mesh = plsc.VectorSubcoreMesh(core_axis_name="c", subcore_axis_name="s")

    @functools.partial(
        pl.kernel,
        out_type=jax.ShapeDtypeStruct((B, C, PTK), jnp.float32),
        mesh=mesh,
        compiler_params=pltpu.CompilerParams(needs_layout_passes=False),
        scratch_types=[
            pltpu.VMEM((PTK,), jnp.int32),
            pltpu.VMEM((PS,), jnp.float32),
            pltpu.VMEM((PTK,), jnp.float32),
        ],
    )
    def gather_kernel(idx_hbm, src_hbm, out_hbm, idx_v, row_v, out_v):
        wid = lax.axis_index("s") * ncores + lax.axis_index("c")
        b = wid // gpb
        g = wid % gpb
        pltpu.sync_copy(idx_hbm.at[b], idx_v)

        def chan_body(i, carry):
            ch = g + i * gpb

            @pl.when(ch < C)
            def _():
                pltpu.sync_copy(src_hbm.at[b, ch], row_v)

                def vec_body(j, carry2):
                    base = j * 64
                    for u in range(4):
                        ivec = idx_v[pl.ds(base + u * 16, 16)]
                        out_v[pl.ds(base + u * 16, 16)] = plsc.load_gather(
                            row_v, [ivec])
                    return carry2

                lax.fori_loop(0, PTK // 64, vec_body, 0)
                pltpu.sync_copy(out_v, out_hbm.at[b, ch])
            return carry

        lax.fori_loop(0, nch, chan_body, 0)

    return gather_kernel(idxf, Source)


def kernel(Target, Source, s_num):
    B, C, PT = Target.shape
    idx = _knn_topk(Target[:, 4:, :], Source[:, 4:, :])
    idx = idx + (jnp.asarray(s_num, dtype=idx.dtype) - K)
    feat = _sc_gather(idx.reshape(B, PT * K), Source)
    return feat.reshape(B, C, PT, K)


# speedup vs baseline: 53.9067x; 1.0341x over previous
import functools

import jax
import jax.numpy as jnp
from jax import lax
from jax.experimental import pallas as pl
from jax.experimental.pallas import tpu as pltpu
from jax.experimental.pallas import tpu_sc as plsc

BT = 256
K = 4


def _dist_tile(q_ref, s_ref):
    q = q_ref[0]
    s = s_ref[0]
    qn = jnp.sum(q * q, axis=0)
    sn = jnp.sum(s * s, axis=0)
    qs = jax.lax.dot_general(q, s, (((0,), (0,)), ((), ())),
                             preferred_element_type=jnp.float32)
    return (qn[:, None] + sn[None, :]) - 2.0 * qs


def _topk_from(d, idx_ref):
    ps = d.shape[1]
    lane = jax.lax.broadcasted_iota(jnp.int32, d.shape, 1).astype(jnp.float32)
    cols = []
    for j in range(K):
        m = jnp.min(d, axis=1, keepdims=True)
        cand = jnp.where(d == m, lane, jnp.float32(ps))
        amin = jnp.min(cand, axis=1, keepdims=True)
        cols.append(amin)
        if j < K - 1:
            d = jnp.where(lane == amin, jnp.float32(jnp.inf), d)
    idxf = jnp.concatenate(cols, axis=1)
    idx_ref[0] = idxf.astype(jnp.int32)


def _topk_body(q_ref, s_ref, idx_ref):
    _topk_from(_dist_tile(q_ref, s_ref), idx_ref)


def _knn_topk(Tq, S):
    B, d, PT = Tq.shape
    PS = S.shape[2]
    grid = (B, PT // BT)
    return pl.pallas_call(
        _topk_body,
        grid=grid,
        in_specs=[
            pl.BlockSpec((1, d, BT), lambda b, p: (b, 0, p)),
            pl.BlockSpec((1, d, PS), lambda b, p: (b, 0, 0)),
        ],
        out_specs=pl.BlockSpec((1, BT, K), lambda b, p: (b, p, 0)),
        out_shape=jax.ShapeDtypeStruct((B, PT, K), jnp.int32),
    )(Tq, S)


def _sc_gather(idxf, Source):
    B, C, PS = Source.shape
    PTK = idxf.shape[1]
    ncores, nsub = 2, 16
    nw = ncores * nsub
    gpb = nw // B
    nch = -(-C // gpb)
    mesh = plsc.VectorSubcoreMesh(core_axis_name="c", subcore_axis_name="s")

    @functools.partial(
        pl.kernel,
        out_type=jax.ShapeDtypeStruct((B, C, PTK), jnp.float32),
        mesh=mesh,
        compiler_params=pltpu.CompilerParams(needs_layout_passes=False),
        scratch_types=[
            pltpu.VMEM((PTK,), jnp.int32),
            pltpu.VMEM((PS,), jnp.float32),
            pltpu.VMEM((PS,), jnp.float32),
            pltpu.VMEM((PTK,), jnp.float32),
            pltpu.VMEM((PTK,), jnp.float32),
            pltpu.SemaphoreType.DMA,
            pltpu.SemaphoreType.DMA,
            pltpu.SemaphoreType.DMA,
            pltpu.SemaphoreType.DMA,
        ],
    )
    def gather_kernel(idx_hbm, src_hbm, out_hbm, idx_v,
                      row0, row1, out0, out1,
                      sin0, sin1, sout0, sout1):
        wid = lax.axis_index("s") * ncores + lax.axis_index("c")
        b = wid // gpb
        g = wid % gpb
        rows = (row0, row1)
        outs = (out0, out1)
        sins = (sin0, sin1)
        souts = (sout0, sout1)
        pltpu.sync_copy(idx_hbm.at[b], idx_v)

        def in_copy(i, buf):
            return pltpu.make_async_copy(
                src_hbm.at[b, g + i * gpb], rows[buf], sins[buf])

        def out_copy(i, buf):
            return pltpu.make_async_copy(
                outs[buf], out_hbm.at[b, g + i * gpb], souts[buf])

        @pl.when(g < C)
        def _():
            in_copy(0, 0).start()

        for i in range(nch):
            cur = i % 2
            ch = g + i * gpb

            if i + 1 < nch:
                @pl.when(ch + gpb < C)
                def _(i=i, cur=cur):
                    in_copy(i + 1, 1 - cur).start()

            @pl.when(ch < C)
            def _(i=i, cur=cur, ch=ch):
                in_copy(i, cur).wait()

                def vec_body(j, carry):
                    base = j * 128
                    for u in range(8):
                        ivec = idx_v[pl.ds(base + u * 16, 16)]
                        outs[cur][pl.ds(base + u * 16, 16)] = plsc.load_gather(
                            rows[cur], [ivec])
                    return carry

                lax.fori_loop(0, PTK // 128, vec_body, 0)
                if i >= 2:
                    out_copy(i - 2, cur).wait()
                out_copy(i, cur).start()

        for i in (nch - 2, nch - 1):
            @pl.when(g + i * gpb < C)
            def _(i=i):
                out_copy(i, i % 2).wait()

    return gather_kernel(idxf, Source)


def kernel(Target, Source, s_num):
    B, C, PT = Target.shape
    idx = _knn_topk(Target[:, 4:, :], Source[:, 4:, :])
    idx = idx + (jnp.asarray(s_num, dtype=idx.dtype) - K)
    feat = _sc_gather(idx.reshape(B, PT * K), Source)
    return feat.reshape(B, C, PT, K)


# speedup vs baseline: 55.3275x; 1.0264x over previous
import functools

import jax
import jax.numpy as jnp
from jax import lax
from jax.experimental import pallas as pl
from jax.experimental.pallas import tpu as pltpu
from jax.experimental.pallas import tpu_sc as plsc

BT = 512
K = 4


def _dist_tile(q_ref, s_ref):
    q = q_ref[0]
    s = s_ref[0]
    qn = jnp.sum(q * q, axis=0)
    sn = jnp.sum(s * s, axis=0)
    qs = jax.lax.dot_general(q, s, (((0,), (0,)), ((), ())),
                             preferred_element_type=jnp.float32)
    return (qn[:, None] + sn[None, :]) - 2.0 * qs


def _topk_from(d, idx_ref):
    ps = d.shape[1]
    lane = jax.lax.broadcasted_iota(jnp.int32, d.shape, 1).astype(jnp.float32)
    cols = []
    for j in range(K):
        m = jnp.min(d, axis=1, keepdims=True)
        cand = jnp.where(d == m, lane, jnp.float32(ps))
        amin = jnp.min(cand, axis=1, keepdims=True)
        cols.append(amin)
        if j < K - 1:
            d = jnp.where(lane == amin, jnp.float32(jnp.inf), d)
    idxf = jnp.concatenate(cols, axis=1)
    idx_ref[0] = idxf.astype(jnp.int32)


def _topk_body(q_ref, s_ref, idx_ref):
    _topk_from(_dist_tile(q_ref, s_ref), idx_ref)


def _knn_topk(Tq, S):
    B, d, PT = Tq.shape
    PS = S.shape[2]
    grid = (B, PT // BT)
    return pl.pallas_call(
        _topk_body,
        grid=grid,
        in_specs=[
            pl.BlockSpec((1, d, BT), lambda b, p: (b, 0, p)),
            pl.BlockSpec((1, d, PS), lambda b, p: (b, 0, 0)),
        ],
        out_specs=pl.BlockSpec((1, BT, K), lambda b, p: (b, p, 0)),
        out_shape=jax.ShapeDtypeStruct((B, PT, K), jnp.int32),
    )(Tq, S)


def _sc_gather(idxf, Source):
    B, C, PS = Source.shape
    PTK = idxf.shape[1]
    ncores, nsub = 2, 16
    nw = ncores * nsub
    gpb = nw // B
    nch = -(-C // gpb)
    mesh = plsc.VectorSubcoreMesh(core_axis_name="c", subcore_axis_name="s")

    @functools.partial(
        pl.kernel,
        out_type=jax.ShapeDtypeStruct((B, C, PTK), jnp.float32),
        mesh=mesh,
        compiler_params=pltpu.CompilerParams(needs_layout_passes=False),
        scratch_types=[
            pltpu.VMEM((PTK,), jnp.int32),
            pltpu.VMEM((PS,), jnp.float32),
            pltpu.VMEM((PS,), jnp.float32),
            pltpu.VMEM((PTK,), jnp.float32),
            pltpu.VMEM((PTK,), jnp.float32),
            pltpu.SemaphoreType.DMA,
            pltpu.SemaphoreType.DMA,
            pltpu.SemaphoreType.DMA,
            pltpu.SemaphoreType.DMA,
        ],
    )
    def gather_kernel(idx_hbm, src_hbm, out_hbm, idx_v,
                      row0, row1, out0, out1,
                      sin0, sin1, sout0, sout1):
        wid = lax.axis_index("s") * ncores + lax.axis_index("c")
        b = wid // gpb
        g = wid % gpb
        rows = (row0, row1)
        outs = (out0, out1)
        sins = (sin0, sin1)
        souts = (sout0, sout1)
        pltpu.sync_copy(idx_hbm.at[b], idx_v)

        def in_copy(i, buf):
            return pltpu.make_async_copy(
                src_hbm.at[b, g + i * gpb], rows[buf], sins[buf])

        def out_copy(i, buf):
            return pltpu.make_async_copy(
                outs[buf], out_hbm.at[b, g + i * gpb], souts[buf])

        @pl.when(g < C)
        def _():
            in_copy(0, 0).start()

        for i in range(nch):
            cur = i % 2
            ch = g + i * gpb

            if i + 1 < nch:
                @pl.when(ch + gpb < C)
                def _(i=i, cur=cur):
                    in_copy(i + 1, 1 - cur).start()

            @pl.when(ch < C)
            def _(i=i, cur=cur, ch=ch):
                in_copy(i, cur).wait()

                def vec_body(j, carry):
                    base = j * 128
                    for u in range(8):
                        ivec = idx_v[pl.ds(base + u * 16, 16)]
                        outs[cur][pl.ds(base + u * 16, 16)] = plsc.load_gather(
                            rows[cur], [ivec])
                    return carry

                lax.fori_loop(0, PTK // 128, vec_body, 0)
                if i >= 2:
                    out_copy(i - 2, cur).wait()
                out_copy(i, cur).start()

        for i in (nch - 2, nch - 1):
            @pl.when(g + i * gpb < C)
            def _(i=i):
                out_copy(i, i % 2).wait()

    return gather_kernel(idxf, Source)


def kernel(Target, Source, s_num):
    B, C, PT = Target.shape
    idx = _knn_topk(Target[:, 4:, :], Source[:, 4:, :])
    idx = idx + (jnp.asarray(s_num, dtype=idx.dtype) - K)
    feat = _sc_gather(idx.reshape(B, PT * K), Source)
    return feat.reshape(B, C, PT, K)
